# Initial kernel scaffold; baseline (speedup 1.0000x reference)
import jax
import jax.numpy as jnp
from jax.experimental import pallas as pl


def kernel(distances1, distances2):
    raise NotImplementedError("write your pallas kernel here")


# speedup vs baseline: 2102.1029x; 2102.1029x over previous
import functools
import jax
import jax.numpy as jnp
from jax import lax
from jax.experimental import pallas as pl
from jax.experimental.pallas import tpu as pltpu
from jax.experimental.pallas import tpu_sc as plsc

_N = 512
_L = 16
_NCHUNK = _N // _L
_INF = float("inf")
_IBIG = 2**31 - 1


def _take(x, perm):
    dnums = lax.GatherDimensionNumbers(
        offset_dims=(), collapsed_slice_dims=(0,), start_index_map=(0,))
    return lax.gather(x, perm[:, None], dnums, (1,),
                      mode=lax.GatherScatterMode.PROMISE_IN_BOUNDS)


def _prim_body(comb_hbm, out_hbm, comb_sh, bestw, bestlin, bestsrc, rowbuf, acc):
    c = lax.axis_index("c")
    s = lax.axis_index("s")

    @pl.when(s == 0)
    def _work():
        pltpu.sync_copy(comb_hbm.at[c], comb_sh)

        pltpu.sync_copy(comb_sh.at[0, pl.ds(0, _N)], bestw)
        for k in range(_NCHUNK):
            ids = lax.iota(jnp.int32, _L) + jnp.int32(_L * k)
            bestlin[pl.ds(_L * k, _L)] = ids
            bestsrc[pl.ds(_L * k, _L)] = jnp.zeros((_L,), jnp.int32)
        w0 = bestw[pl.ds(0, _L)]
        bestw[pl.ds(0, _L)] = jnp.where(
            lax.iota(jnp.int32, _L) == 0, _INF, w0)
        acc[...] = jnp.zeros((_L,), jnp.float32)

        def _round(i, carry):
            mw = bestw[pl.ds(0, _L)]
            ml = bestlin[pl.ds(0, _L)]
            mv = lax.iota(jnp.int32, _L)
            ms = bestsrc[pl.ds(0, _L)]
            for k in range(1, _NCHUNK):
                w = bestw[pl.ds(_L * k, _L)]
                l = bestlin[pl.ds(_L * k, _L)]
                sv = bestsrc[pl.ds(_L * k, _L)]
                idv = lax.iota(jnp.int32, _L) + jnp.int32(_L * k)
                b = (w < mw) | ((w == mw) & (l < ml))
                mw = jnp.where(b, w, mw)
                ml = jnp.where(b, l, ml)
                mv = jnp.where(b, idv, mv)
                ms = jnp.where(b, sv, ms)
            lane = lax.iota(jnp.int32, _L)
            for sh in (8, 4, 2, 1):
                perm = lane ^ jnp.int32(sh)
                w2 = _take(mw, perm)
                l2 = _take(ml, perm)
                v2 = _take(mv, perm)
                s2 = _take(ms, perm)
                b = (w2 < mw) | ((w2 == mw) & (l2 < ml))
                mw = jnp.where(b, w2, mw)
                ml = jnp.where(b, l2, ml)
                mv = jnp.where(b, v2, mv)
                ms = jnp.where(b, s2, ms)
            vstar = mv[0]

            pltpu.sync_copy(comb_sh.at[vstar], rowbuf)

            base_v = jnp.int32(_N) + (ms & jnp.int32(-_L))
            dchunk = rowbuf[pl.ds(base_v[0], _L)]
            acc[...] += jnp.where(
                lane == (ms & jnp.int32(_L - 1)), dchunk, jnp.float32(0.0))

            for k in range(_NCHUNK):
                ids = lax.iota(jnp.int32, _L) + jnp.int32(_L * k)
                w = bestw[pl.ds(_L * k, _L)]
                l = bestlin[pl.ds(_L * k, _L)]
                sv = bestsrc[pl.ds(_L * k, _L)]
                cw = rowbuf[pl.ds(_L * k, _L)]
                a = jnp.minimum(ids, mv)
                bx = jnp.maximum(ids, mv)
                cl = a * jnp.int32(_N) - ((a * (a - 1)) >> 1) + (bx - a)
                upd = ((cw < w) | ((cw == w) & (cl < l))) & (w < _INF)
                w2 = jnp.where(upd, cw, w)
                l2 = jnp.where(upd, cl, l)
                s2 = jnp.where(upd, mv, sv)
                w2 = jnp.where(ids == mv, _INF, w2)
                bestw[pl.ds(_L * k, _L)] = w2
                bestlin[pl.ds(_L * k, _L)] = l2
                bestsrc[pl.ds(_L * k, _L)] = s2
            return carry

        lax.fori_loop(0, _N - 1, _round, jnp.int32(0))
        pltpu.sync_copy(acc, out_hbm.at[c])


@jax.jit
def _toposig_loss(comb):
    mesh = plsc.VectorSubcoreMesh(core_axis_name="c", subcore_axis_name="s")
    run = functools.partial(
        pl.kernel,
        mesh=mesh,
        out_type=jax.ShapeDtypeStruct((2, _L), jnp.float32),
        scratch_types=[
            pltpu.VMEM_SHARED((_N, 2 * _N), jnp.float32),
            pltpu.VMEM((_N,), jnp.float32),
            pltpu.VMEM((_N,), jnp.int32),
            pltpu.VMEM((_N,), jnp.int32),
            pltpu.VMEM((2 * _N,), jnp.float32),
            pltpu.VMEM((_L,), jnp.float32),
        ],
    )(_prim_body)
    return run(comb)


def kernel(distances1, distances2):
    d1 = distances1.astype(jnp.float32)
    d2 = distances2.astype(jnp.float32)
    i = jnp.arange(_N, dtype=jnp.int32)[:, None]
    j = jnp.arange(_N, dtype=jnp.int32)[None, :]
    upper = j > i
    s1 = jnp.where(upper, d1, d1.T)
    s1 = jnp.where(i == j, _INF, s1)
    s2 = jnp.where(upper, d2, d2.T)
    s2 = jnp.where(i == j, _INF, s2)
    diff = d1 - d2
    dsq = jnp.where(upper, diff, diff.T) ** 2
    comb = jnp.stack([
        jnp.concatenate([s1, dsq], axis=1),
        jnp.concatenate([s2, dsq], axis=1),
    ])
    out = _toposig_loss(comb)
    return out.sum()
